# baseline (speedup 1.0000x reference)
import jax
import jax.numpy as jnp
from jax.experimental import pallas as pl
from jax.experimental.pallas import tpu as pltpu

EMBED = 512
EXPERTS = 16
TOPK = 4
HEADS = 16
DEPTH = 2
NUM_CLASSES = 1000
POOL = 14
B, P, C, H, W = 16, 16, 3, 56, 56
N = B * P
HD = EMBED // HEADS


def _ln(x, g, b):
    m = jnp.mean(x, axis=-1, keepdims=True)
    v = jnp.mean((x - m) ** 2, axis=-1, keepdims=True)
    return (x - m) / jnp.sqrt(v + 1e-6) * g + b


def _stem_body(x_ref, w_ref, b_ref, o_ref):
    xb = x_ref[...]
    nc = xb.shape[0] // (C * H)
    t1 = jnp.concatenate(
        [jnp.sum(xb[:, 4 * j:4 * j + 4], axis=1, keepdims=True)
         for j in range(POOL)], axis=1)
    t4 = t1.reshape(nc * C, POOL, 4, POOL)
    s = (t4[:, :, 0, :] + t4[:, :, 1, :] + t4[:, :, 2, :] + t4[:, :, 3, :])
    s = s * (1.0 / 16.0)
    s = s.reshape(nc, C * POOL, POOL)
    feats = jnp.concatenate([s[:, k, :] for k in range(C * POOL)], axis=-1)
    o_ref[...] = jnp.dot(feats, w_ref[...],
                         preferred_element_type=jnp.float32) + b_ref[...]


def _stem(x2, w2, b):
    chunks = 8
    nc = N // chunks
    return pl.pallas_call(
        _stem_body,
        grid=(chunks,),
        in_specs=[
            pl.BlockSpec((nc * C * H, W), lambda i: (i, 0)),
            pl.BlockSpec((C * POOL * POOL, EMBED), lambda i: (0, 0)),
            pl.BlockSpec((1, EMBED), lambda i: (0, 0)),
        ],
        out_specs=pl.BlockSpec((nc, EMBED), lambda i: (i, 0)),
        out_shape=jax.ShapeDtypeStruct((N, EMBED), jnp.float32),
    )(x2, w2, b)


def _attn_body(h_ref, g_ref, bln_ref, wqkv_ref, bqkv_ref, wo_ref, bo_ref,
               o_ref):
    h = h_ref[0]
    y = _ln(h, g_ref[...], bln_ref[...])
    qkv = jnp.dot(y, wqkv_ref[...], preferred_element_type=jnp.float32)
    qkv = qkv + bqkv_ref[...]

    def hm(off):
        return jnp.concatenate(
            [qkv[:, off + HD * h:off + HD * (h + 1)] for h in range(HEADS)],
            axis=0)

    q2 = hm(0)
    k2 = hm(EMBED)
    v2 = hm(2 * EMBED)
    s = jnp.dot(q2, k2.T, preferred_element_type=jnp.float32)
    s = s / jnp.sqrt(jnp.float32(HD))
    hi = jax.lax.broadcasted_iota(jnp.int32, (HEADS * P, HEADS * P), 0) // P
    hj = jax.lax.broadcasted_iota(jnp.int32, (HEADS * P, HEADS * P), 1) // P
    s = jnp.where(hi == hj, s, -1e30)
    s = s - jnp.max(s, axis=-1, keepdims=True)
    e = jnp.exp(s)
    att = e / jnp.sum(e, axis=-1, keepdims=True)
    o2 = jnp.dot(att, v2, preferred_element_type=jnp.float32)
    o = jnp.concatenate([o2[P * h:P * (h + 1), :] for h in range(HEADS)],
                        axis=1)
    o_ref[0] = h + jnp.dot(o, wo_ref[...],
                           preferred_element_type=jnp.float32) + bo_ref[...]


def _attn(h3, g, bln, wqkv, bqkv, wo, bo):
    return pl.pallas_call(
        _attn_body,
        grid=(B,),
        in_specs=[
            pl.BlockSpec((1, P, EMBED), lambda b: (b, 0, 0)),
            pl.BlockSpec((1, EMBED), lambda b: (0, 0)),
            pl.BlockSpec((1, EMBED), lambda b: (0, 0)),
            pl.BlockSpec((EMBED, 3 * EMBED), lambda b: (0, 0)),
            pl.BlockSpec((1, 3 * EMBED), lambda b: (0, 0)),
            pl.BlockSpec((EMBED, EMBED), lambda b: (0, 0)),
            pl.BlockSpec((1, EMBED), lambda b: (0, 0)),
        ],
        out_specs=pl.BlockSpec((1, P, EMBED), lambda b: (b, 0, 0)),
        out_shape=jax.ShapeDtypeStruct((B, P, EMBED), jnp.float32),
    )(h3, g, bln, wqkv, bqkv, wo, bo)


def _moe_body(h_ref, g_ref, bln_ref, wg_ref, we1_ref, be1_ref, we2_ref,
              be2_ref, o_ref, y_s, comb_s, acc_s):
    e = pl.program_id(0)

    @pl.when(e == 0)
    def _init():
        h = h_ref[...]
        y = _ln(h, g_ref[...], bln_ref[...])
        y_s[...] = y
        logits = jnp.dot(y, wg_ref[...], preferred_element_type=jnp.float32)
        logits = logits - jnp.max(logits, axis=-1, keepdims=True)
        ex = jnp.exp(logits)
        probs = ex / jnp.sum(ex, axis=-1, keepdims=True)
        lane = jax.lax.broadcasted_iota(jnp.int32, (N, EXPERTS), 1)
        r = jnp.zeros((N, EXPERTS), jnp.int32)
        for j in range(EXPERTS):
            pj = probs[:, j:j + 1]
            beats = (pj > probs) | ((pj == probs) & (j < lane))
            r = r + beats.astype(jnp.int32)
        sel = (r < TOPK).astype(jnp.float32)
        cw = probs * sel
        comb_s[...] = cw / jnp.sum(cw, axis=-1, keepdims=True)

    y = y_s[...]
    t = jnp.dot(y, we1_ref[0], preferred_element_type=jnp.float32)
    h1 = jax.nn.gelu(t + be1_ref[0])
    h2 = jnp.dot(h1, we2_ref[0], preferred_element_type=jnp.float32)
    h2 = h2 + be2_ref[0]
    lane = jax.lax.broadcasted_iota(jnp.int32, (N, EXPERTS), 1)
    ce = jnp.sum(comb_s[...] * (lane == e).astype(jnp.float32), axis=1,
                 keepdims=True)
    contrib = (ce.astype(jnp.bfloat16).astype(jnp.float32) *
               h2.astype(jnp.bfloat16).astype(jnp.float32))

    @pl.when(e == 0)
    def _first():
        acc_s[...] = contrib

    @pl.when(e > 0)
    def _rest():
        acc_s[...] += contrib

    @pl.when(e == EXPERTS - 1)
    def _final():
        o_ref[...] = h_ref[...] + acc_s[...]


def _moe(h2d, g, bln, wg, we1, be1, we2, be2):
    return pl.pallas_call(
        _moe_body,
        grid=(EXPERTS,),
        in_specs=[
            pl.BlockSpec((N, EMBED), lambda e: (0, 0)),
            pl.BlockSpec((1, EMBED), lambda e: (0, 0)),
            pl.BlockSpec((1, EMBED), lambda e: (0, 0)),
            pl.BlockSpec((EMBED, EXPERTS), lambda e: (0, 0)),
            pl.BlockSpec((1, EMBED, EMBED), lambda e: (e, 0, 0)),
            pl.BlockSpec((1, 1, EMBED), lambda e: (e, 0, 0)),
            pl.BlockSpec((1, EMBED, EMBED), lambda e: (e, 0, 0)),
            pl.BlockSpec((1, 1, EMBED), lambda e: (e, 0, 0)),
        ],
        out_specs=pl.BlockSpec((N, EMBED), lambda e: (0, 0)),
        out_shape=jax.ShapeDtypeStruct((N, EMBED), jnp.float32),
        scratch_shapes=[
            pltpu.VMEM((N, EMBED), jnp.float32),
            pltpu.VMEM((N, EXPERTS), jnp.float32),
            pltpu.VMEM((N, EMBED), jnp.float32),
        ],
    )(h2d, g, bln, wg, we1, be1.reshape(EXPERTS, 1, EMBED),
      we2, be2.reshape(EXPERTS, 1, EMBED))


def _head_body(h_ref, g_ref, bln_ref, w_ref, b_ref, o_ref):
    y = _ln(h_ref[...], g_ref[...], bln_ref[...])
    y3 = y.reshape(B, P, EMBED)
    pooled = y3[:, 0, :]
    for p in range(1, P):
        pooled = pooled + y3[:, p, :]
    pooled = pooled * (1.0 / P)
    o_ref[...] = jnp.dot(pooled, w_ref[...],
                         preferred_element_type=jnp.float32) + b_ref[...]


def _head(h2d, g, bln, w, b):
    return pl.pallas_call(
        _head_body,
        out_shape=jax.ShapeDtypeStruct((B, NUM_CLASSES), jnp.float32),
    )(h2d, g, bln, w, b)


def kernel(x, params):
    x2 = x.reshape(N * C * H, W)
    feats = _stem(x2, params['W_stem'], params['b_stem'].reshape(1, EMBED))

    h = feats
    for p in params['layers']:
        h3 = _attn(h.reshape(B, P, EMBED),
                   p['ln1_g'].reshape(1, EMBED), p['ln1_b'].reshape(1, EMBED),
                   p['Wqkv'], p['bqkv'].reshape(1, 3 * EMBED),
                   p['Wo'], p['bo'].reshape(1, EMBED))
        h = _moe(h3.reshape(N, EMBED),
                 p['ln2_g'].reshape(1, EMBED), p['ln2_b'].reshape(1, EMBED),
                 p['Wg'], p['We1'], p['be1'], p['We2'], p['be2'])

    return _head(h, params['lnf_g'].reshape(1, EMBED),
                 params['lnf_b'].reshape(1, EMBED),
                 params['W_head'], params['b_head'].reshape(1, NUM_CLASSES))


# speedup vs baseline: 1.9614x; 1.9614x over previous
import jax
import jax.numpy as jnp
from jax.experimental import pallas as pl
from jax.experimental.pallas import tpu as pltpu

EMBED = 512
EXPERTS = 16
TOPK = 4
HEADS = 16
DEPTH = 2
NUM_CLASSES = 1000
POOL = 14
B, P, C, H, W = 16, 16, 3, 56, 56
N = B * P
HD = EMBED // HEADS


def _ln(x, g, b):
    m = jnp.mean(x, axis=-1, keepdims=True)
    v = jnp.mean((x - m) ** 2, axis=-1, keepdims=True)
    return (x - m) / jnp.sqrt(v + 1e-6) * g + b


def _stem_body(x_ref, m_ref, w_ref, b_ref, o_ref):
    xb = x_ref[...]
    nc = xb.shape[0] // (C * H)
    x4 = xb.reshape(nc * C, POOL, 4, W)
    hs = x4[:, :, 0, :] + x4[:, :, 1, :] + x4[:, :, 2, :] + x4[:, :, 3, :]
    hs = hs.reshape(nc * C * POOL, W)
    s = jnp.dot(hs, m_ref[...], preferred_element_type=jnp.float32,
                precision=jax.lax.Precision.HIGHEST)
    s3 = s.reshape(nc, C * POOL, POOL)
    w3 = w_ref[...].reshape(C * POOL, POOL, EMBED)
    feats = b_ref[...]
    for k in range(C * POOL):
        feats = feats + jnp.dot(s3[:, k, :], w3[k],
                                preferred_element_type=jnp.float32)
    o_ref[...] = feats


def _stem(x2, w2, b):
    chunks = 8
    nc = N // chunks
    m = (jnp.arange(W)[:, None] // 4 == jnp.arange(POOL)[None, :])
    m = m.astype(jnp.float32) / 16.0
    return pl.pallas_call(
        _stem_body,
        grid=(chunks,),
        in_specs=[
            pl.BlockSpec((nc * C * H, W), lambda i: (i, 0)),
            pl.BlockSpec((W, POOL), lambda i: (0, 0)),
            pl.BlockSpec((C * POOL * POOL, EMBED), lambda i: (0, 0)),
            pl.BlockSpec((1, EMBED), lambda i: (0, 0)),
        ],
        out_specs=pl.BlockSpec((nc, EMBED), lambda i: (i, 0)),
        out_shape=jax.ShapeDtypeStruct((N, EMBED), jnp.float32),
    )(x2, m, w2, b)


def _attn_body(h_ref, g_ref, bln_ref, wqkv_ref, bqkv_ref, wo_ref, bo_ref,
               o_ref):
    h = h_ref[0]
    y = _ln(h, g_ref[...], bln_ref[...])
    qkv = jnp.dot(y, wqkv_ref[...], preferred_element_type=jnp.float32)
    qkv = qkv + bqkv_ref[...]

    qkvT = qkv.T
    q3 = qkvT[:EMBED].reshape(HEADS, HD, P)
    k3 = qkvT[EMBED:2 * EMBED].reshape(HEADS, HD, P)
    v3 = qkvT[2 * EMBED:].reshape(HEADS, HD, P)
    s = jax.lax.dot_general(q3, k3, (((1,), (1,)), ((0,), (0,))),
                            preferred_element_type=jnp.float32)
    s = s / jnp.sqrt(jnp.float32(HD))
    s = s - jnp.max(s, axis=-1, keepdims=True)
    ex = jnp.exp(s)
    att = ex / jnp.sum(ex, axis=-1, keepdims=True)
    o3 = jax.lax.dot_general(v3, att, (((2,), (2,)), ((0,), (0,))),
                             preferred_element_type=jnp.float32)
    wo = wo_ref[...]
    acc = h + bo_ref[...]
    for hh in range(HEADS):
        acc = acc + jax.lax.dot_general(
            o3[hh], wo[HD * hh:HD * (hh + 1), :], (((0,), (0,)), ((), ())),
            preferred_element_type=jnp.float32)
    o_ref[0] = acc


def _attn(h3, g, bln, wqkv, bqkv, wo, bo):
    return pl.pallas_call(
        _attn_body,
        grid=(B,),
        in_specs=[
            pl.BlockSpec((1, P, EMBED), lambda b: (b, 0, 0)),
            pl.BlockSpec((1, EMBED), lambda b: (0, 0)),
            pl.BlockSpec((1, EMBED), lambda b: (0, 0)),
            pl.BlockSpec((EMBED, 3 * EMBED), lambda b: (0, 0)),
            pl.BlockSpec((1, 3 * EMBED), lambda b: (0, 0)),
            pl.BlockSpec((EMBED, EMBED), lambda b: (0, 0)),
            pl.BlockSpec((1, EMBED), lambda b: (0, 0)),
        ],
        out_specs=pl.BlockSpec((1, P, EMBED), lambda b: (b, 0, 0)),
        out_shape=jax.ShapeDtypeStruct((B, P, EMBED), jnp.float32),
    )(h3, g, bln, wqkv, bqkv, wo, bo)


def _moe_body(h_ref, g_ref, bln_ref, wg_ref, we1_ref, be1_ref, we2_ref,
              be2_ref, o_ref, y_s, comb_s, acc_s):
    e = pl.program_id(0)

    @pl.when(e == 0)
    def _init():
        h = h_ref[...]
        y = _ln(h, g_ref[...], bln_ref[...])
        y_s[...] = y
        logits = jnp.dot(y, wg_ref[...], preferred_element_type=jnp.float32)
        logits = logits - jnp.max(logits, axis=-1, keepdims=True)
        ex = jnp.exp(logits)
        probs = ex / jnp.sum(ex, axis=-1, keepdims=True)
        lane = jax.lax.broadcasted_iota(jnp.int32, (N, EXPERTS), 1)
        r = jnp.zeros((N, EXPERTS), jnp.int32)
        for j in range(EXPERTS):
            pj = probs[:, j:j + 1]
            beats = (pj > probs) | ((pj == probs) & (j < lane))
            r = r + beats.astype(jnp.int32)
        sel = (r < TOPK).astype(jnp.float32)
        cw = probs * sel
        comb_s[...] = cw / jnp.sum(cw, axis=-1, keepdims=True)

    y = y_s[...]
    t = jnp.dot(y, we1_ref[0], preferred_element_type=jnp.float32)
    h1 = jax.nn.gelu(t + be1_ref[0])
    h2 = jnp.dot(h1, we2_ref[0], preferred_element_type=jnp.float32)
    h2 = h2 + be2_ref[0]
    lane = jax.lax.broadcasted_iota(jnp.int32, (N, EXPERTS), 1)
    ce = jnp.sum(comb_s[...] * (lane == e).astype(jnp.float32), axis=1,
                 keepdims=True)
    contrib = (ce.astype(jnp.bfloat16).astype(jnp.float32) *
               h2.astype(jnp.bfloat16).astype(jnp.float32))

    @pl.when(e == 0)
    def _first():
        acc_s[...] = contrib

    @pl.when(e > 0)
    def _rest():
        acc_s[...] += contrib

    @pl.when(e == EXPERTS - 1)
    def _final():
        o_ref[...] = h_ref[...] + acc_s[...]


def _moe(h2d, g, bln, wg, we1, be1, we2, be2):
    return pl.pallas_call(
        _moe_body,
        grid=(EXPERTS,),
        in_specs=[
            pl.BlockSpec((N, EMBED), lambda e: (0, 0)),
            pl.BlockSpec((1, EMBED), lambda e: (0, 0)),
            pl.BlockSpec((1, EMBED), lambda e: (0, 0)),
            pl.BlockSpec((EMBED, EXPERTS), lambda e: (0, 0)),
            pl.BlockSpec((1, EMBED, EMBED), lambda e: (e, 0, 0)),
            pl.BlockSpec((1, 1, EMBED), lambda e: (e, 0, 0)),
            pl.BlockSpec((1, EMBED, EMBED), lambda e: (e, 0, 0)),
            pl.BlockSpec((1, 1, EMBED), lambda e: (e, 0, 0)),
        ],
        out_specs=pl.BlockSpec((N, EMBED), lambda e: (0, 0)),
        out_shape=jax.ShapeDtypeStruct((N, EMBED), jnp.float32),
        scratch_shapes=[
            pltpu.VMEM((N, EMBED), jnp.float32),
            pltpu.VMEM((N, EXPERTS), jnp.float32),
            pltpu.VMEM((N, EMBED), jnp.float32),
        ],
    )(h2d, g, bln, wg, we1, be1.reshape(EXPERTS, 1, EMBED),
      we2, be2.reshape(EXPERTS, 1, EMBED))


def _head_body(h_ref, g_ref, bln_ref, w_ref, b_ref, o_ref):
    y = _ln(h_ref[...], g_ref[...], bln_ref[...])
    y3 = y.reshape(B, P, EMBED)
    pooled = y3[:, 0, :]
    for p in range(1, P):
        pooled = pooled + y3[:, p, :]
    pooled = pooled * (1.0 / P)
    o_ref[...] = jnp.dot(pooled, w_ref[...],
                         preferred_element_type=jnp.float32) + b_ref[...]


def _head(h2d, g, bln, w, b):
    return pl.pallas_call(
        _head_body,
        out_shape=jax.ShapeDtypeStruct((B, NUM_CLASSES), jnp.float32),
    )(h2d, g, bln, w, b)


def kernel(x, params):
    x2 = x.reshape(N * C * H, W)
    feats = _stem(x2, params['W_stem'], params['b_stem'].reshape(1, EMBED))

    h = feats
    for p in params['layers']:
        h3 = _attn(h.reshape(B, P, EMBED),
                   p['ln1_g'].reshape(1, EMBED), p['ln1_b'].reshape(1, EMBED),
                   p['Wqkv'], p['bqkv'].reshape(1, 3 * EMBED),
                   p['Wo'], p['bo'].reshape(1, EMBED))
        h = _moe(h3.reshape(N, EMBED),
                 p['ln2_g'].reshape(1, EMBED), p['ln2_b'].reshape(1, EMBED),
                 p['Wg'], p['We1'], p['be1'], p['We2'], p['be2'])

    return _head(h, params['lnf_g'].reshape(1, EMBED),
                 params['lnf_b'].reshape(1, EMBED),
                 params['W_head'], params['b_head'].reshape(1, NUM_CLASSES))
